# baseline (speedup 1.0000x reference)
import functools

import jax
import jax.numpy as jnp
from jax import lax
from jax.experimental import pallas as pl
from jax.experimental.pallas import tpu as pltpu
from jax.experimental.pallas import tpu_sc as plsc

N_KP = 16
INPUT_MIN = -1.0
INPUT_MAX = 1.0
STEP = (INPUT_MAX - INPUT_MIN) / (N_KP - 1)
INV_STEP = (N_KP - 1) / (INPUT_MAX - INPUT_MIN)
LANES = 16

N_WORKERS = 32
CHUNK = 16384
_MAGIC = 12582912.0 + INV_STEP - 0.5


def _sc_body(per_worker, n_chunks,
             x_hbm, raw_hbm, y_hbm, raw_v, dv, c0_v, c1_v,
             xb0, xb1, yb0, yb1, si0, si1, so0, so1):
    nc = lax.axis_size("c")
    wid = lax.axis_index("s") * nc + lax.axis_index("c")
    base = wid * per_worker

    xbufs, ybufs = (xb0, xb1), (yb0, yb1)
    sin, sout = (si0, si1), (so0, so1)

    def start_in(cc, b):
        pltpu.async_copy(
            x_hbm.at[pl.ds(base + cc * CHUNK, CHUNK)], xbufs[b], sin[b])

    start_in(0, 0)
    start_in(1, 1)

    pltpu.sync_copy(raw_hbm, raw_v)
    r = raw_v[...]
    e = jnp.exp(-jnp.abs(r))
    z = e * (1.0 - e * (0.5 - e * (1.0 / 3.0)))
    w = 1.0 + e
    for _ in range(3):
        z = z - 1.0 + w * jnp.exp(-z)
    d = jnp.maximum(r, 0.0) + z
    cs = jnp.cumsum(d)
    denom = jnp.max(cs) + 1e-6
    io = lax.iota(jnp.int32, 16)
    dv[...] = d
    dn = plsc.load_gather(dv, [(io + 1) & 15])
    dn = jnp.where(io < N_KP - 1, dn, 0.0)
    y = cs / denom
    ynext = (cs + dn) / denom
    kx = INPUT_MIN + io.astype(jnp.float32) * STEP
    c1 = (ynext - y) / (STEP + 1e-8)
    c0_v[...] = y - c1 * kx
    c1_v[...] = c1

    def compute(xb, yb):
        @plsc.parallel_loop(0, CHUNK, step=LANES, unroll=4)
        def _vec(i):
            xv = xb[pl.ds(i, LANES)]
            v = jnp.minimum(jnp.maximum(xv, INPUT_MIN), INPUT_MAX)
            w = v * INV_STEP + _MAGIC
            j = plsc.bitcast(w, jnp.int32) & 0xFFFF
            a = plsc.load_gather(c0_v, [j])
            b = plsc.load_gather(c1_v, [j])
            yb[pl.ds(i, LANES)] = a + b * v

    def wait_in(b):
        pltpu.make_async_copy(
            x_hbm.at[pl.ds(0, CHUNK)], xbufs[b], sin[b]).wait()

    def wait_out(b):
        pltpu.make_async_copy(
            ybufs[b], y_hbm.at[pl.ds(0, CHUNK)], sout[b]).wait()

    @pl.loop(0, n_chunks, step=2)
    def _pair(c):
        for b in (0, 1):
            cc = c + b
            wait_in(b)

            @pl.when(c >= 2)
            def _():
                wait_out(b)

            compute(xbufs[b], ybufs[b])
            pltpu.async_copy(
                ybufs[b], y_hbm.at[pl.ds(base + cc * CHUNK, CHUNK)], sout[b])

            @pl.when(c + 2 < n_chunks)
            def _():
                start_in(cc + 2, b)

    wait_out(0)
    wait_out(1)


def kernel(x, keypoint_y_raw):
    n = x.size
    per_worker = n // N_WORKERS
    n_chunks = per_worker // CHUNK

    mesh = plsc.VectorSubcoreMesh(core_axis_name="c", subcore_axis_name="s")
    sc = pl.kernel(
        functools.partial(_sc_body, per_worker, n_chunks),
        out_type=jax.ShapeDtypeStruct((n,), jnp.float32),
        mesh=mesh,
        scratch_types=[
            pltpu.VMEM((N_KP,), jnp.float32),
            pltpu.VMEM((N_KP,), jnp.float32),
            pltpu.VMEM((N_KP,), jnp.float32),
            pltpu.VMEM((N_KP,), jnp.float32),
            pltpu.VMEM((CHUNK,), jnp.float32),
            pltpu.VMEM((CHUNK,), jnp.float32),
            pltpu.VMEM((CHUNK,), jnp.float32),
            pltpu.VMEM((CHUNK,), jnp.float32),
            pltpu.SemaphoreType.DMA,
            pltpu.SemaphoreType.DMA,
            pltpu.SemaphoreType.DMA,
            pltpu.SemaphoreType.DMA,
        ],
        compiler_params=pltpu.CompilerParams(needs_layout_passes=False),
    )
    return sc(x, keypoint_y_raw)
